# baseline (speedup 1.0000x reference)
import functools
import math

import jax
import jax.numpy as jnp
from jax import lax
from jax.experimental import pallas as pl
from jax.experimental.pallas import tpu as pltpu
from jax.experimental.pallas import tpu_sc as plsc

NUM_EMBEDDINGS = 1000000
D_MODEL = 64
SCALE = math.sqrt(D_MODEL)

B_ROWS = 4096
B_COLS = 50
N_TOTAL = B_ROWS * B_COLS

_INFO = plsc.get_sparse_core_info()
NC = _INFO.num_cores
NS = _INFO.num_subcores
NW = NC * NS
LANES = _INFO.num_lanes

PER_W = N_TOTAL // NW
SUB = 128
CHUNK = 256
K_SUB = CHUNK // SUB
N_CHUNKS = PER_W // CHUNK
GROUPS = CHUNK // LANES


def _body(table_hbm, idx_hbm, out_hbm, idx_v, gbuf, hbuf, rows_v, stage_v, sem):
    wid = lax.axis_index("s") * NC + lax.axis_index("c")
    base = wid * PER_W

    pltpu.sync_copy(idx_hbm.at[pl.ds(base, PER_W)], idx_v)

    def split_idx(k, _):
        v = idx_v[pl.ds(k * LANES, LANES)]
        gbuf[pl.ds(k * LANES, LANES)] = lax.shift_right_logical(v, 1)
        hbuf[pl.ds(k * LANES, LANES)] = (v & 1) * D_MODEL
        return 0

    lax.fori_loop(0, PER_W // LANES, split_idx, 0)

    iota = lax.iota(jnp.int32, LANES)

    def do_chunk(c, _):
        off = c * CHUNK
        for j in range(K_SUB):
            pltpu.async_copy(table_hbm.at[gbuf.at[pl.ds(off + j * SUB, SUB)]],
                             rows_v.at[pl.ds(j * SUB, SUB)], sem)
        for j in range(K_SUB):
            pltpu.make_async_copy(
                table_hbm.at[gbuf.at[pl.ds(off + j * SUB, SUB)]],
                rows_v.at[pl.ds(j * SUB, SUB)], sem).wait()

        def do_group(g, _):
            hv = hbuf[pl.ds(off + g * LANES, LANES)]
            gb = g * LANES
            for l in range(LANES):
                hsp = hv.at[jnp.full((LANES,), l, jnp.int32)].get(
                    mode="promise_in_bounds")
                rowv = jnp.full((LANES,), gb + l, jnp.int32)
                for j in range(D_MODEL // LANES):
                    colv = hsp + (iota + j * LANES)
                    vals = plsc.load_gather(rows_v, [rowv, colv])
                    stage_v[gb + l, pl.ds(j * LANES, LANES)] = vals * SCALE
            return 0

        lax.fori_loop(0, GROUPS, do_group, 0)
        pltpu.sync_copy(stage_v, out_hbm.at[pl.ds(base + off, CHUNK)])
        return 0

    lax.fori_loop(0, N_CHUNKS, do_chunk, 0)


@jax.jit
def _embed(table_p, idx1d):
    mesh = plsc.VectorSubcoreMesh(core_axis_name="c", subcore_axis_name="s")
    kern = pl.kernel(
        _body,
        out_type=jax.ShapeDtypeStruct((N_TOTAL, D_MODEL), jnp.float32),
        mesh=mesh,
        scratch_types=[
            pltpu.VMEM((PER_W,), jnp.int32),
            pltpu.VMEM((PER_W,), jnp.int32),
            pltpu.VMEM((PER_W,), jnp.int32),
            pltpu.VMEM((CHUNK, 2 * D_MODEL), jnp.float32),
            pltpu.VMEM((CHUNK, D_MODEL), jnp.float32),
            pltpu.SemaphoreType.DMA,
        ],
        compiler_params=pltpu.CompilerParams(use_tc_tiling_on_sc=True,
                                             needs_layout_passes=False),
    )
    return kern(table_p, idx1d)


def kernel(inputs, table):
    table_p = table.reshape(NUM_EMBEDDINGS // 2, 2 * D_MODEL)
    idx1d = inputs.reshape(-1).astype(jnp.int32)
    out = _embed(table_p, idx1d)
    return out.reshape(B_ROWS, B_COLS, D_MODEL)
